# baseline (speedup 1.0000x reference)
import jax
import jax.numpy as jnp
from jax.experimental import pallas as pl
from jax.experimental.pallas import tpu as pltpu

_B, _N, _F, _FE, _FG, _H, _FOUT, _STEPS = 8, 128, 128, 16, 128, 128, 128, 4
_TI = 32
_NT = _N // _TI


def _dot(a, b):
    return jax.lax.dot_general(a, b, (((1,), (0,)), ((), ())),
                               preferred_element_type=jnp.float32)


def _body(node_ref, edge_ref, graph_ref, adj_ref, wen_ref, wee_ref, weg_ref,
          m1_ref, m2_ref, me_ref, mg_ref, o1_ref, o2_ref, wd_ref,
          out_ref, me_s):
    bf16 = jnp.bfloat16
    node_h = _dot(node_ref[0], wen_ref[...])
    wfe = _dot(wee_ref[...], me_ref[...]).astype(bf16)
    wg = _dot(weg_ref[...], mg_ref[...])
    mg = _dot(graph_ref[0], wg)

    edge = edge_ref[0]
    for t in range(_NT):
        blk = edge[t * _TI:(t + 1) * _TI].astype(bf16)
        me_s[t * _TI:(t + 1) * _TI] = _dot(
            blk.reshape(_TI * _N, _FE), wfe).astype(bf16).reshape(_TI, _N, _H)

    m1w, m2w, o1w, wd = m1_ref[...], m2_ref[...], o1_ref[...], wd_ref[...]
    a1 = _dot(node_h, m1w[:_H])
    a2 = _dot(node_h, m2w[:_H]) + mg
    o1a = _dot(node_h, o1w[:_H])
    adj = adj_ref[0].astype(bf16)

    hid = None
    for _ in range(_STEPS):
        if hid is None:
            m1, m2, hl = a1, a2, o1a
        else:
            m1 = a1 + _dot(hid, m1w[_H:])
            m2 = a2 + _dot(hid, m2w[_H:])
            hl = o1a + _dot(hid, o1w[_H:])
        m1b, m2b = m1.astype(bf16), m2.astype(bf16)
        aggs = []
        for t in range(_NT):
            sl = slice(t * _TI, (t + 1) * _TI)
            msgs = jnp.maximum(me_s[sl] + m1b[sl][:, None, :] + m2b[None, :, :],
                               bf16(0.0))
            aggs.append(jax.lax.dot_general(
                adj[sl], msgs, (((1,), (1,)), ((0,), (0,))),
                preferred_element_type=jnp.float32))
        agg = jnp.concatenate(aggs, axis=0)
        hid = jnp.maximum(hl + _dot(agg, o2_ref[...]), 0.0)

    out_ref[0] = _dot(node_h, wd[:_H]) + _dot(hid, wd[_H:])


def kernel(node_fts, edge_fts, graph_fts, adj, W_enc_node, W_enc_edge,
           W_enc_graph, M1, M2, Me, Mg, O1, O2, W_dec):
    graph3 = graph_fts.reshape(_B, 1, _FG)
    wspec = lambda *shape: pl.BlockSpec(shape, lambda b: (0,) * len(shape))
    return pl.pallas_call(
        _body,
        grid=(_B,),
        in_specs=[
            pl.BlockSpec((1, _N, _F), lambda b: (b, 0, 0)),
            pl.BlockSpec((1, _N, _N, _FE), lambda b: (b, 0, 0, 0)),
            pl.BlockSpec((1, 1, _FG), lambda b: (b, 0, 0)),
            pl.BlockSpec((1, _N, _N), lambda b: (b, 0, 0)),
            wspec(_F, _H),
            wspec(_FE, _H),
            wspec(_FG, _H),
            wspec(2 * _H, _H),
            wspec(2 * _H, _H),
            wspec(_H, _H),
            wspec(_H, _H),
            wspec(2 * _H, _H),
            wspec(_H, _H),
            wspec(2 * _H, _FOUT),
        ],
        out_specs=pl.BlockSpec((1, _N, _FOUT), lambda b: (b, 0, 0)),
        out_shape=jax.ShapeDtypeStruct((_B, _N, _FOUT), jnp.float32),
        scratch_shapes=[pltpu.VMEM((_N, _N, _H), jnp.bfloat16)],
        compiler_params=pltpu.CompilerParams(
            dimension_semantics=("parallel",)),
    )(node_fts, edge_fts, graph3, adj, W_enc_node, W_enc_edge, W_enc_graph,
      M1, M2, Me, Mg, O1, O2, W_dec)


# speedup vs baseline: 1.5058x; 1.5058x over previous
import jax
import jax.numpy as jnp
from jax.experimental import pallas as pl
from jax.experimental.pallas import tpu as pltpu

_B, _N, _F, _FE, _FG, _H, _FOUT, _STEPS = 8, 128, 128, 16, 128, 128, 128, 4
_TI = 32
_NT = _N // _TI


def _dot(a, b):
    return jax.lax.dot_general(a, b, (((1,), (0,)), ((), ())),
                               preferred_element_type=jnp.float32)


def _body(node_ref, edge_ref, graph_ref, adj_ref, wen_ref, wee_ref, weg_ref,
          m1_ref, m2_ref, me_ref, mg_ref, o1_ref, o2_ref, wd_ref,
          out_ref, me_s):
    bf16 = jnp.bfloat16
    node_h = _dot(node_ref[0], wen_ref[...])
    wfe = _dot(wee_ref[...], me_ref[...]).astype(bf16)
    wg = _dot(weg_ref[...], mg_ref[...])
    mg = _dot(graph_ref[0], wg)

    edge = edge_ref[0]
    wfe_b = jnp.broadcast_to(wfe[None], (_TI, _FE, _H))
    for t in range(_NT):
        blk = edge[t * _TI:(t + 1) * _TI]
        me_s[t * _TI:(t + 1) * _TI] = jax.lax.dot_general(
            blk, wfe_b, (((1,), (1,)), ((0,), (0,))),
            preferred_element_type=jnp.float32).astype(bf16)

    m1w, m2w, o1w, wd = m1_ref[...], m2_ref[...], o1_ref[...], wd_ref[...]
    a1 = _dot(node_h, m1w[:_H])
    a2 = _dot(node_h, m2w[:_H]) + mg
    o1a = _dot(node_h, o1w[:_H])
    adj = adj_ref[0].astype(bf16)

    hid = None
    for _ in range(_STEPS):
        if hid is None:
            m1, m2, hl = a1, a2, o1a
        else:
            m1 = a1 + _dot(hid, m1w[_H:])
            m2 = a2 + _dot(hid, m2w[_H:])
            hl = o1a + _dot(hid, o1w[_H:])
        m1b, m2b = m1.astype(bf16), m2.astype(bf16)
        aggs = []
        for t in range(_NT):
            sl = slice(t * _TI, (t + 1) * _TI)
            msgs = jnp.maximum(me_s[sl] + m1b[sl][:, None, :] + m2b[None, :, :],
                               bf16(0.0))
            aggs.append(jax.lax.dot_general(
                adj[sl], msgs, (((1,), (1,)), ((0,), (0,))),
                preferred_element_type=jnp.float32))
        agg = jnp.concatenate(aggs, axis=0)
        hid = jnp.maximum(hl + _dot(agg, o2_ref[...]), 0.0)

    out_ref[0] = _dot(node_h, wd[:_H]) + _dot(hid, wd[_H:])


def kernel(node_fts, edge_fts, graph_fts, adj, W_enc_node, W_enc_edge,
           W_enc_graph, M1, M2, Me, Mg, O1, O2, W_dec):
    graph3 = graph_fts.reshape(_B, 1, _FG)
    edge_t = edge_fts.astype(jnp.bfloat16).transpose(0, 1, 3, 2)
    wspec = lambda *shape: pl.BlockSpec(shape, lambda b: (0,) * len(shape))
    return pl.pallas_call(
        _body,
        grid=(_B,),
        in_specs=[
            pl.BlockSpec((1, _N, _F), lambda b: (b, 0, 0)),
            pl.BlockSpec((1, _N, _FE, _N), lambda b: (b, 0, 0, 0)),
            pl.BlockSpec((1, 1, _FG), lambda b: (b, 0, 0)),
            pl.BlockSpec((1, _N, _N), lambda b: (b, 0, 0)),
            wspec(_F, _H),
            wspec(_FE, _H),
            wspec(_FG, _H),
            wspec(2 * _H, _H),
            wspec(2 * _H, _H),
            wspec(_H, _H),
            wspec(_H, _H),
            wspec(2 * _H, _H),
            wspec(_H, _H),
            wspec(2 * _H, _FOUT),
        ],
        out_specs=pl.BlockSpec((1, _N, _FOUT), lambda b: (b, 0, 0)),
        out_shape=jax.ShapeDtypeStruct((_B, _N, _FOUT), jnp.float32),
        scratch_shapes=[pltpu.VMEM((_N, _N, _H), jnp.bfloat16)],
        compiler_params=pltpu.CompilerParams(
            dimension_semantics=("parallel",)),
    )(node_fts, edge_t, graph3, adj, W_enc_node, W_enc_edge, W_enc_graph,
      M1, M2, Me, Mg, O1, O2, W_dec)
